# baseline (speedup 1.0000x reference)
import functools

import jax
import jax.numpy as jnp
from jax import lax
from jax.experimental import pallas as pl
from jax.experimental.pallas import tpu as pltpu
from jax.experimental.pallas import tpu_sc as plsc

SEQ_LEN = 1024
EMBED_DIM = 16

_info = plsc.get_sparse_core_info()
_NC, _NS, _L = _info.num_cores, _info.num_subcores, _info.num_lanes
_NW = _NC * _NS
_BPW = SEQ_LEN // _NW
_NGROUPS = _BPW // _L

_mesh = plsc.VectorSubcoreMesh(core_axis_name="c", subcore_axis_name="s")


@functools.partial(
    pl.kernel,
    mesh=_mesh,
    out_type=jax.ShapeDtypeStruct((SEQ_LEN * EMBED_DIM,), jnp.float32),
    compiler_params=pltpu.CompilerParams(needs_layout_passes=False),
    scratch_types=[
        pltpu.VMEM((_BPW,), jnp.int32),
        pltpu.VMEM((128 * EMBED_DIM,), jnp.float32),
        pltpu.VMEM((_BPW * EMBED_DIM,), jnp.float32),
    ],
)
def _embed_add(idx_hbm, tok_hbm, pos_hbm, out_hbm, idx_v, tok_v, out_v):
    wid = lax.axis_index("s") * _NC + lax.axis_index("c")
    base = wid * _BPW
    pltpu.sync_copy(idx_hbm.at[pl.ds(base, _BPW)], idx_v)
    pltpu.sync_copy(tok_hbm, tok_v)
    pltpu.sync_copy(pos_hbm.at[pl.ds(base * EMBED_DIM, _BPW * EMBED_DIM)], out_v)
    for g in range(_NGROUPS):
        tok_base = idx_v[pl.ds(g * _L, _L)] * EMBED_DIM
        out_base = (lax.iota(jnp.int32, _L) + (g * _L)) * EMBED_DIM
        for d in range(EMBED_DIM):
            vals = plsc.load_gather(tok_v, [tok_base + d])
            plsc.addupdate_scatter(out_v, [out_base + d], vals)
    pltpu.sync_copy(out_v, out_hbm.at[pl.ds(base * EMBED_DIM, _BPW * EMBED_DIM)])


def kernel(encoded, token_table, pos_table):
    return _embed_add(
        encoded.astype(jnp.int32),
        token_table.reshape(-1),
        pos_table.reshape(-1),
    ).reshape(SEQ_LEN, EMBED_DIM)


# speedup vs baseline: 1.0344x; 1.0344x over previous
import functools

import jax
import jax.numpy as jnp
from jax import lax
from jax.experimental import pallas as pl
from jax.experimental.pallas import tpu as pltpu
from jax.experimental.pallas import tpu_sc as plsc

SEQ_LEN = 1024
EMBED_DIM = 16
VOCAB = 128

_info = plsc.get_sparse_core_info()
_NC, _NS, _L = _info.num_cores, _info.num_subcores, _info.num_lanes
_NW = _NC * _NS
_BPW = SEQ_LEN // _NW
_NGROUPS = _BPW // _L

_mesh = plsc.VectorSubcoreMesh(core_axis_name="c", subcore_axis_name="s")


@functools.partial(
    pl.kernel,
    mesh=_mesh,
    out_type=jax.ShapeDtypeStruct((SEQ_LEN * EMBED_DIM,), jnp.float32),
    compiler_params=pltpu.CompilerParams(needs_layout_passes=False),
    scratch_types=[
        pltpu.VMEM((_BPW,), jnp.int32),
        pltpu.VMEM((VOCAB * EMBED_DIM,), jnp.float32),
        pltpu.VMEM((_BPW * EMBED_DIM,), jnp.float32),
        pltpu.SemaphoreType.DMA,
        pltpu.SemaphoreType.DMA,
        pltpu.SemaphoreType.DMA,
    ],
)
def _embed_add(idx_hbm, tok_hbm, pos_hbm, out_hbm, idx_v, tok_v, out_v,
               idx_sem, tok_sem, pos_sem):
    wid = lax.axis_index("s") * _NC + lax.axis_index("c")
    base = wid * _BPW
    idx_cp = pltpu.async_copy(idx_hbm.at[pl.ds(base, _BPW)], idx_v, idx_sem)
    tok_cp = pltpu.async_copy(tok_hbm, tok_v, tok_sem)
    pos_cp = pltpu.async_copy(
        pos_hbm.at[pl.ds(base * EMBED_DIM, _BPW * EMBED_DIM)], out_v, pos_sem)
    idx_cp.wait()
    tok_cp.wait()
    pos_cp.wait()
    for g in range(_NGROUPS):
        tok_base = idx_v[pl.ds(g * _L, _L)] * EMBED_DIM
        out_base = (lax.iota(jnp.int32, _L) + (g * _L)) * EMBED_DIM
        for d in range(EMBED_DIM):
            vals = plsc.load_gather(tok_v, [tok_base + d])
            plsc.addupdate_scatter(out_v, [out_base + d], vals)
    pltpu.sync_copy(out_v, out_hbm.at[pl.ds(base * EMBED_DIM, _BPW * EMBED_DIM)])


def kernel(encoded, token_table, pos_table):
    return _embed_add(
        encoded.astype(jnp.int32),
        token_table.reshape(-1),
        pos_table.reshape(-1),
    ).reshape(SEQ_LEN, EMBED_DIM)
